# baseline (speedup 1.0000x reference)
import functools

import jax
import jax.numpy as jnp
import numpy as np
from jax import lax
from jax.experimental import pallas as pl
from jax.experimental.pallas import tpu as pltpu
from jax.experimental.pallas import tpu_sc as plsc

_DIM_LEN = 768
_GROUP = 4

_B = 2
_SPLIT = 8
_R = _B * _DIM_LEN * _SPLIT
_D = 224 * 224 // _SPLIT
_GROWS = _GROUP * _SPLIT
_NG = _R // _GROWS
_NC = 2
_NS = 16
_NW = _NC * _NS
_GPW = _NG // _NW


def _src_group_rows() -> np.ndarray:
    with jax.ensure_compile_time_eval():
        perm = np.asarray(jax.random.permutation(jax.random.key(1), _DIM_LEN // _GROUP))
    rows = (np.arange(_B)[:, None] * _DIM_LEN + perm[None, :] * _GROUP).reshape(-1)
    return (rows * _SPLIT).astype(np.int32)


_SRC_GROUPS = _src_group_rows()


def _permute_rows(x2, src):
    mesh = plsc.ScalarSubcoreMesh(axis_name="c", num_cores=_NC)
    gpc = _NG // _NC

    @functools.partial(
        pl.kernel,
        mesh=mesh,
        out_type=jax.ShapeDtypeStruct((_R, _D), jnp.float32),
        scratch_types=[
            pltpu.SMEM((_NG,), jnp.int32),
            pltpu.SemaphoreType.DMA,
        ],
    )
    def k(x_hbm, src_hbm, out_hbm, idx_s, sem):
        base = lax.axis_index("c") * gpc
        pltpu.sync_copy(src_hbm, idx_s)

        def fire(j, carry):
            srow = pl.multiple_of(idx_s[base + j], _GROWS)
            pltpu.async_copy(
                x_hbm.at[pl.ds(srow, _GROWS)],
                out_hbm.at[pl.ds((base + j) * _GROWS, _GROWS)],
                sem,
            )
            return carry

        def drain(j, carry):
            srow = pl.multiple_of(idx_s[base + j], _GROWS)
            pltpu.make_async_copy(
                x_hbm.at[pl.ds(srow, _GROWS)],
                out_hbm.at[pl.ds((base + j) * _GROWS, _GROWS)],
                sem,
            ).wait()
            return carry

        lax.fori_loop(0, gpc, fire, 0)
        lax.fori_loop(0, gpc, drain, 0)

    return k(x2, src)


def kernel(x):
    x2 = x.reshape(_R, _D)
    src = jnp.asarray(_SRC_GROUPS)
    out2 = _permute_rows(x2, src)
    return out2.reshape(x.shape)
